# Initial kernel scaffold; baseline (speedup 1.0000x reference)
import jax
import jax.numpy as jnp
from jax.experimental import pallas as pl


def kernel(x, edge_index, W1, b1, W2, b2, W3, b3, Wc, bc):
    raise NotImplementedError("write your pallas kernel here")


# speedup vs baseline: 56.9544x; 56.9544x over previous
import functools

import jax
import jax.numpy as jnp
from jax import lax
from jax.experimental import pallas as pl
from jax.experimental.pallas import tpu as pltpu
from jax.experimental.pallas import tpu_sc as plsc

N = 10000
NP = 10240
E = 320000
F = 4
FLAT = F * NP
NC, NS = 2, 16
NW = NC * NS
EPW = E // NW
L = 16
BLK = 1024
G = NP // BLK

_mesh = plsc.VectorSubcoreMesh(core_axis_name="c", subcore_axis_name="s")


@functools.partial(
    pl.kernel,
    out_type=jax.ShapeDtypeStruct((NW, NP), jnp.float32),
    mesh=_mesh,
    scratch_types=[
        pltpu.VMEM((EPW,), jnp.int32),
        pltpu.VMEM((NP,), jnp.float32),
    ],
    compiler_params=pltpu.CompilerParams(needs_layout_passes=False),
)
def _deg_kernel(col_hbm, out_hbm, col_v, acc_v):
    wid = lax.axis_index("s") * NC + lax.axis_index("c")
    pltpu.sync_copy(col_hbm.at[pl.ds(wid * EPW, EPW)], col_v)
    zeros = jnp.zeros((L,), jnp.float32)

    def zbody(i, carry):
        acc_v[pl.ds(i * L, L)] = zeros
        return carry

    lax.fori_loop(0, NP // L, zbody, 0)
    ones = jnp.ones((L,), jnp.float32)

    def ebody(i, carry):
        cols = col_v[pl.ds(i * L, L)]
        plsc.addupdate_scatter(acc_v, [cols], ones)
        return carry

    lax.fori_loop(0, EPW // L, ebody, 0)
    pltpu.sync_copy(acc_v, out_hbm.at[wid])


@functools.partial(
    pl.kernel,
    out_type=jax.ShapeDtypeStruct((NW, FLAT), jnp.float32),
    mesh=_mesh,
    scratch_types=[
        pltpu.VMEM((EPW,), jnp.int32),
        pltpu.VMEM((EPW,), jnp.int32),
        pltpu.VMEM((FLAT,), jnp.float32),
        pltpu.VMEM((FLAT,), jnp.float32),
    ],
    compiler_params=pltpu.CompilerParams(needs_layout_passes=False),
)
def _agg_kernel(y_hbm, row_hbm, col_hbm, zero_hbm, out_hbm, row_v, col_v, y_v, acc_v):
    wid = lax.axis_index("s") * NC + lax.axis_index("c")
    base = wid * EPW
    pltpu.sync_copy(y_hbm, y_v)
    pltpu.sync_copy(zero_hbm, acc_v)
    pltpu.sync_copy(row_hbm.at[pl.ds(base, EPW)], row_v)
    pltpu.sync_copy(col_hbm.at[pl.ds(base, EPW)], col_v)

    def ebody(i, carry):
        rows = row_v[pl.ds(i * L, L)]
        cols = col_v[pl.ds(i * L, L)]
        for f in range(F):
            off = f * NP
            v = plsc.load_gather(y_v, [rows + off])
            plsc.addupdate_scatter(acc_v, [cols + off], v)
        return carry

    lax.fori_loop(0, EPW // L, ebody, 0)
    pltpu.sync_copy(acc_v, out_hbm.at[wid])


def _prep_body(x_ref, w1t_ref, degp_ref, y1_ref, dinv_ref):
    xw_t = lax.dot_general(w1t_ref[...], x_ref[...], (((1,), (1,)), ((), ())),
                           preferred_element_type=jnp.float32)
    deg = 1.0 + jnp.sum(degp_ref[...], axis=0, keepdims=True)
    dinv = lax.rsqrt(deg)
    dinv_ref[...] = dinv
    y1_ref[...] = xw_t * jnp.broadcast_to(dinv, (F, BLK))


def _prep(x_pad, w1t, deg_p):
    return pl.pallas_call(
        _prep_body,
        grid=(G,),
        in_specs=[
            pl.BlockSpec((BLK, 128), lambda i: (i, 0)),
            pl.BlockSpec((F, 128), lambda i: (0, 0)),
            pl.BlockSpec((NW, BLK), lambda i: (0, i)),
        ],
        out_specs=[
            pl.BlockSpec((F, BLK), lambda i: (0, i)),
            pl.BlockSpec((1, BLK), lambda i: (0, i)),
        ],
        out_shape=[
            jax.ShapeDtypeStruct((F, NP), jnp.float32),
            jax.ShapeDtypeStruct((1, NP), jnp.float32),
        ],
    )(x_pad, w1t, deg_p)


def _mid_body(p_ref, y_ref, dinv_ref, wt_ref, b_ref, ynext_ref):
    agg = jnp.sum(p_ref[...], axis=0) + y_ref[...]
    dinv = jnp.broadcast_to(dinv_ref[...], (F, BLK))
    h = jnp.tanh(dinv * agg + b_ref[...])
    ynext_ref[...] = dinv * lax.dot_general(
        wt_ref[...], h, (((1,), (0,)), ((), ())),
        preferred_element_type=jnp.float32)


def _mid(p, y, dinv, wt, bmat):
    return pl.pallas_call(
        _mid_body,
        grid=(G,),
        in_specs=[
            pl.BlockSpec((NW, F, BLK), lambda i: (0, 0, i)),
            pl.BlockSpec((F, BLK), lambda i: (0, i)),
            pl.BlockSpec((1, BLK), lambda i: (0, i)),
            pl.BlockSpec((F, F), lambda i: (0, 0)),
            pl.BlockSpec((F, BLK), lambda i: (0, i)),
        ],
        out_specs=pl.BlockSpec((F, BLK), lambda i: (0, i)),
        out_shape=jax.ShapeDtypeStruct((F, NP), jnp.float32),
    )(p, y, dinv, wt, bmat)


def _fin_body(p_ref, y_ref, dinv_ref, b_ref, wct_ref, bc_ref, ht_ref, rt_ref):
    agg = jnp.sum(p_ref[...], axis=0) + y_ref[...]
    dinv = jnp.broadcast_to(dinv_ref[...], (F, BLK))
    h = jnp.tanh(dinv * agg + b_ref[...])
    ht_ref[...] = h
    rt_ref[...] = lax.dot_general(
        wct_ref[...], h, (((1,), (0,)), ((), ())),
        preferred_element_type=jnp.float32) + bc_ref[...]


def _fin(p, y, dinv, bmat, wct, bcmat):
    return pl.pallas_call(
        _fin_body,
        grid=(G,),
        in_specs=[
            pl.BlockSpec((NW, F, BLK), lambda i: (0, 0, i)),
            pl.BlockSpec((F, BLK), lambda i: (0, i)),
            pl.BlockSpec((1, BLK), lambda i: (0, i)),
            pl.BlockSpec((F, BLK), lambda i: (0, i)),
            pl.BlockSpec((8, F), lambda i: (0, 0)),
            pl.BlockSpec((8, BLK), lambda i: (0, i)),
        ],
        out_specs=[
            pl.BlockSpec((F, BLK), lambda i: (0, i)),
            pl.BlockSpec((8, BLK), lambda i: (0, i)),
        ],
        out_shape=[
            jax.ShapeDtypeStruct((F, NP), jnp.float32),
            jax.ShapeDtypeStruct((8, NP), jnp.float32),
        ],
    )(p, y, dinv, bmat, wct, bcmat)


def kernel(x, edge_index, W1, b1, W2, b2, W3, b3, Wc, bc):
    ei = edge_index.astype(jnp.int32)
    row, col = ei[0], ei[1]
    x_pad = jnp.pad(x, ((0, NP - N), (0, 0)))
    zeros_flat = jnp.zeros((FLAT,), jnp.float32)

    w1t = W1.T
    w2t = W2.T
    w3t = jnp.pad(W3.T, ((0, F - W3.shape[1]), (0, 0)))
    wct = jnp.pad(Wc.T, ((0, 0), (0, F - Wc.shape[0])))
    b1m = jnp.broadcast_to(b1[:, None], (F, NP))
    b2m = jnp.broadcast_to(b2[:, None], (F, NP))
    b3m = jnp.broadcast_to(jnp.pad(b3, (0, F - b3.shape[0]))[:, None], (F, NP))
    bcm = jnp.broadcast_to(bc[:, None], (8, NP))

    deg_p = _deg_kernel(col)
    y1, dinv = _prep(x_pad, w1t, deg_p)

    p1 = _agg_kernel(y1.reshape(FLAT), row, col, zeros_flat).reshape(NW, F, NP)
    y2 = _mid(p1, y1, dinv, w2t, b1m)

    p2 = _agg_kernel(y2.reshape(FLAT), row, col, zeros_flat).reshape(NW, F, NP)
    y3 = _mid(p2, y2, dinv, w3t, b2m)

    p3 = _agg_kernel(y3.reshape(FLAT), row, col, zeros_flat).reshape(NW, F, NP)
    ht, rt = _fin(p3, y3, dinv, b3m, wct, bcm)

    h = ht[:2].T[:N]
    result = rt.T[:N]
    return (result, h)


# speedup vs baseline: 75.4593x; 1.3249x over previous
import functools

import jax
import jax.numpy as jnp
from jax import lax
from jax.experimental import pallas as pl
from jax.experimental.pallas import tpu as pltpu
from jax.experimental.pallas import tpu_sc as plsc

N = 10000
NP = 10240
E = 320000
F = 4
FLAT = F * NP
NC, NS = 2, 16
NW = NC * NS
EPW = E // NW
L = 16
U = 2
BLK = 1024
G = NP // BLK

_mesh = plsc.VectorSubcoreMesh(core_axis_name="c", subcore_axis_name="s")
_sc_params = pltpu.CompilerParams(needs_layout_passes=False)


@functools.partial(
    pl.kernel,
    out_type=jax.ShapeDtypeStruct((NW, NP), jnp.float32),
    mesh=_mesh,
    scratch_types=[
        pltpu.VMEM((EPW,), jnp.int32),
        pltpu.VMEM((NP,), jnp.float32),
        pltpu.SemaphoreType.DMA,
    ],
    compiler_params=_sc_params,
)
def _deg_kernel(ei_hbm, out_hbm, col_v, acc_v, sem):
    wid = lax.axis_index("s") * NC + lax.axis_index("c")
    cp = pltpu.async_copy(ei_hbm.at[pl.ds(E + wid * EPW, EPW)], col_v, sem)
    zeros = jnp.zeros((L,), jnp.float32)

    def zbody(i, carry):
        for u in range(4):
            acc_v[pl.ds((i * 4 + u) * L, L)] = zeros
        return carry

    lax.fori_loop(0, NP // (L * 4), zbody, 0)
    cp.wait()
    ones = jnp.ones((L,), jnp.float32)

    def ebody(i, carry):
        for u in range(4):
            cols = col_v[pl.ds((i * 4 + u) * L, L)]
            plsc.addupdate_scatter(acc_v, [cols], ones)
        return carry

    lax.fori_loop(0, EPW // (L * 4), ebody, 0)
    pltpu.sync_copy(acc_v, out_hbm.at[wid])


@functools.partial(
    pl.kernel,
    out_type=jax.ShapeDtypeStruct((NW, FLAT), jnp.float32),
    mesh=_mesh,
    scratch_types=[
        pltpu.VMEM((EPW,), jnp.int32),
        pltpu.VMEM((EPW,), jnp.int32),
        pltpu.VMEM((FLAT,), jnp.float32),
        pltpu.VMEM((FLAT,), jnp.float32),
        pltpu.SemaphoreType.DMA,
        pltpu.SemaphoreType.DMA,
        pltpu.SemaphoreType.DMA,
    ],
    compiler_params=_sc_params,
)
def _agg_kernel(y_hbm, ei_hbm, out_hbm, row_v, col_v, y_v, acc_v,
                sem_y, sem_r, sem_c):
    wid = lax.axis_index("s") * NC + lax.axis_index("c")
    base = wid * EPW
    cp_y = pltpu.async_copy(y_hbm, y_v, sem_y)
    cp_r = pltpu.async_copy(ei_hbm.at[pl.ds(base, EPW)], row_v, sem_r)
    cp_c = pltpu.async_copy(ei_hbm.at[pl.ds(E + base, EPW)], col_v, sem_c)
    zeros = jnp.zeros((L,), jnp.float32)

    def zbody(i, carry):
        for u in range(4):
            acc_v[pl.ds((i * 4 + u) * L, L)] = zeros
        return carry

    lax.fori_loop(0, FLAT // (L * 4), zbody, 0)
    cp_y.wait()
    cp_r.wait()
    cp_c.wait()

    def ebody(i, carry):
        b = i * (L * U)
        rs = [row_v[pl.ds(b + u * L, L)] for u in range(U)]
        cs = [col_v[pl.ds(b + u * L, L)] for u in range(U)]
        vals = [plsc.load_gather(y_v, [rs[u] + f * NP])
                for u in range(U) for f in range(F)]
        for u in range(U):
            for f in range(F):
                plsc.addupdate_scatter(acc_v, [cs[u] + f * NP], vals[u * F + f])
        return carry

    lax.fori_loop(0, EPW // (L * U), ebody, 0)
    pltpu.sync_copy(acc_v, out_hbm.at[wid])


def _prep_body(x_ref, w1t_ref, degp_ref, y1_ref, dinv_ref):
    xw_t = lax.dot_general(w1t_ref[...], x_ref[...], (((1,), (1,)), ((), ())),
                           preferred_element_type=jnp.float32)
    deg = 1.0 + jnp.sum(degp_ref[...], axis=0, keepdims=True)
    dinv = lax.rsqrt(deg)
    dinv_ref[...] = dinv
    y1_ref[...] = xw_t * jnp.broadcast_to(dinv, (F, BLK))


def _prep(x, w1t, deg_p):
    return pl.pallas_call(
        _prep_body,
        grid=(G,),
        in_specs=[
            pl.BlockSpec((BLK, 128), lambda i: (i, 0)),
            pl.BlockSpec((F, 128), lambda i: (0, 0)),
            pl.BlockSpec((NW, BLK), lambda i: (0, i)),
        ],
        out_specs=[
            pl.BlockSpec((F, BLK), lambda i: (0, i)),
            pl.BlockSpec((1, BLK), lambda i: (0, i)),
        ],
        out_shape=[
            jax.ShapeDtypeStruct((F, NP), jnp.float32),
            jax.ShapeDtypeStruct((1, NP), jnp.float32),
        ],
    )(x, w1t, deg_p)


def _mid_body(p_ref, y_ref, dinv_ref, wt_ref, b_ref, ynext_ref):
    agg = jnp.sum(p_ref[...], axis=0) + y_ref[...]
    dinv = jnp.broadcast_to(dinv_ref[...], (F, BLK))
    h = jnp.tanh(dinv * agg + b_ref[...])
    ynext_ref[...] = dinv * lax.dot_general(
        wt_ref[...], h, (((1,), (0,)), ((), ())),
        preferred_element_type=jnp.float32)


def _mid(p, y, dinv, wt, bmat):
    return pl.pallas_call(
        _mid_body,
        grid=(G,),
        in_specs=[
            pl.BlockSpec((NW, F, BLK), lambda i: (0, 0, i)),
            pl.BlockSpec((F, BLK), lambda i: (0, i)),
            pl.BlockSpec((1, BLK), lambda i: (0, i)),
            pl.BlockSpec((F, F), lambda i: (0, 0)),
            pl.BlockSpec((F, BLK), lambda i: (0, i)),
        ],
        out_specs=pl.BlockSpec((F, BLK), lambda i: (0, i)),
        out_shape=jax.ShapeDtypeStruct((F, NP), jnp.float32),
    )(p, y, dinv, wt, bmat)


def _fin_body(p_ref, y_ref, dinv_ref, b_ref, wct_ref, bc_ref, h_ref, r_ref):
    agg = jnp.sum(p_ref[...], axis=0) + y_ref[...]
    dinv = jnp.broadcast_to(dinv_ref[...], (F, BLK))
    h = jnp.tanh(dinv * agg + b_ref[...])
    h_ref[...] = lax.transpose(h, (1, 0))
    r = lax.dot_general(wct_ref[...], h, (((1,), (0,)), ((), ())),
                        preferred_element_type=jnp.float32) + bc_ref[...]
    r_ref[...] = lax.transpose(r, (1, 0))


def _fin(p, y, dinv, bmat, wct, bcmat):
    return pl.pallas_call(
        _fin_body,
        grid=(G,),
        in_specs=[
            pl.BlockSpec((NW, F, BLK), lambda i: (0, 0, i)),
            pl.BlockSpec((F, BLK), lambda i: (0, i)),
            pl.BlockSpec((1, BLK), lambda i: (0, i)),
            pl.BlockSpec((F, BLK), lambda i: (0, i)),
            pl.BlockSpec((8, F), lambda i: (0, 0)),
            pl.BlockSpec((8, BLK), lambda i: (0, i)),
        ],
        out_specs=[
            pl.BlockSpec((BLK, F), lambda i: (i, 0)),
            pl.BlockSpec((BLK, 8), lambda i: (i, 0)),
        ],
        out_shape=[
            jax.ShapeDtypeStruct((NP, F), jnp.float32),
            jax.ShapeDtypeStruct((NP, 8), jnp.float32),
        ],
    )(p, y, dinv, bmat, wct, bcmat)


def kernel(x, edge_index, W1, b1, W2, b2, W3, b3, Wc, bc):
    ei = edge_index.astype(jnp.int32).reshape(2 * E)

    w1t = W1.T
    w2t = W2.T
    w3t = jnp.pad(W3.T, ((0, F - W3.shape[1]), (0, 0)))
    wct = jnp.pad(Wc.T, ((0, 0), (0, F - Wc.shape[0])))
    b1m = jnp.broadcast_to(b1[:, None], (F, NP))
    b2m = jnp.broadcast_to(b2[:, None], (F, NP))
    b3m = jnp.broadcast_to(jnp.pad(b3, (0, F - b3.shape[0]))[:, None], (F, NP))
    bcm = jnp.broadcast_to(bc[:, None], (8, NP))

    deg_p = _deg_kernel(ei)
    y1, dinv = _prep(x, w1t, deg_p)

    p1 = _agg_kernel(y1.reshape(FLAT), ei).reshape(NW, F, NP)
    y2 = _mid(p1, y1, dinv, w2t, b1m)

    p2 = _agg_kernel(y2.reshape(FLAT), ei).reshape(NW, F, NP)
    y3 = _mid(p2, y2, dinv, w3t, b2m)

    p3 = _agg_kernel(y3.reshape(FLAT), ei).reshape(NW, F, NP)
    h_nm, r_nm = _fin(p3, y3, dinv, b3m, wct, bcm)

    return (r_nm[:N], h_nm[:N, :2])
